# baseline (speedup 1.0000x reference)
import jax
import jax.numpy as jnp
from jax.experimental import pallas as pl
from jax.experimental.pallas import tpu as pltpu

_B, _L, _CLEN, _E, _DM = 16, 64, 512, 8, 1024
_FIN = 3 * _CLEN
_ROWS = _B * _L


def _moe_kernel(logits_ref, masks_ref, flat_ref, w_ref, b_ref, out_ref, gl_ref):
    e = pl.program_id(0)
    ne = pl.num_programs(0)

    logits = logits_ref[...]
    mask = (masks_ref[...] == 1).astype(jnp.float32)
    m = jnp.max(logits, axis=1, keepdims=True)
    ex = jnp.exp(logits - m)
    sm = ex / jnp.sum(ex, axis=1, keepdims=True)
    g = sm * mask
    g = g / (jnp.sum(g, axis=1, keepdims=True) + 1e-9)

    @pl.when(e == 0)
    def _():
        s = jnp.sum(sm * mask) / _B
        gl_ref[0, 0] = (1.0 - s) * (1.0 - s)

    onehot = (jax.lax.broadcasted_iota(jnp.int32, (_B, _E), 1) == e)
    g_sel = jnp.sum(g * onehot.astype(jnp.float32), axis=1, keepdims=True)
    row_b = jax.lax.broadcasted_iota(jnp.int32, (_ROWS, _B), 0) // _L
    col_b = jax.lax.broadcasted_iota(jnp.int32, (_ROWS, _B), 1)
    P = (row_b == col_b).astype(jnp.float32)
    ge_col = jnp.dot(P, g_sel, preferred_element_type=jnp.float32)

    xge = (ge_col * flat_ref[...]).astype(jnp.bfloat16)
    w = w_ref[0].astype(jnp.bfloat16)
    partial = jnp.dot(xge, w, preferred_element_type=jnp.float32)
    partial = partial + ge_col * b_ref[0]

    @pl.when(e == 0)
    def _():
        out_ref[...] = jnp.zeros_like(out_ref)

    out_ref[...] += partial


def kernel(cycle_curve_data, logits, moe_masks, expert_w, expert_b):
    flat = cycle_curve_data.reshape(_ROWS, _FIN)
    masks = moe_masks.astype(jnp.int32)

    out2d, gl = pl.pallas_call(
        _moe_kernel,
        grid=(_E,),
        in_specs=[
            pl.BlockSpec((_B, _E), lambda e: (0, 0)),
            pl.BlockSpec((_B, _E), lambda e: (0, 0)),
            pl.BlockSpec((_ROWS, _FIN), lambda e: (0, 0)),
            pl.BlockSpec((1, _FIN, _DM), lambda e: (e, 0, 0)),
            pl.BlockSpec((1, 1, _DM), lambda e: (e, 0, 0)),
        ],
        out_specs=[
            pl.BlockSpec((_ROWS, _DM), lambda e: (0, 0)),
            pl.BlockSpec(memory_space=pltpu.SMEM),
        ],
        out_shape=[
            jax.ShapeDtypeStruct((_ROWS, _DM), jnp.float32),
            jax.ShapeDtypeStruct((1, 1), jnp.float32),
        ],
    )(logits, masks, flat, expert_w, expert_b.reshape(_E, 1, _DM))

    final_out = out2d.astype(jnp.bfloat16).reshape(_B, _L, _DM)
    guide_loss = gl.reshape(())
    return final_out, guide_loss


# speedup vs baseline: 1.3059x; 1.3059x over previous
import jax
import jax.numpy as jnp
from jax.experimental import pallas as pl
from jax.experimental.pallas import tpu as pltpu

_B, _L, _CLEN, _E, _DM = 16, 64, 512, 8, 1024
_FIN = 3 * _CLEN
_ROWS = _B * _L


def _moe_kernel(logits_ref, masks_ref, cc_ref, w_ref, b_ref, out_ref, gl_ref,
                flat_ref, acc_ref):
    e = pl.program_id(0)
    ne = pl.num_programs(0)

    @pl.when(e == 0)
    def _():
        for c in range(3):
            xc = cc_ref[:, :, c, :].reshape(_ROWS, _CLEN)
            flat_ref[:, c * _CLEN:(c + 1) * _CLEN] = xc.astype(jnp.bfloat16)

    logits = logits_ref[...]
    mask = (masks_ref[...] == 1).astype(jnp.float32)
    m = jnp.max(logits, axis=1, keepdims=True)
    ex = jnp.exp(logits - m)
    sm = ex / jnp.sum(ex, axis=1, keepdims=True)
    g = sm * mask
    g = g / (jnp.sum(g, axis=1, keepdims=True) + 1e-9)

    @pl.when(e == 0)
    def _():
        s = jnp.sum(sm * mask) / _B
        gl_ref[0, 0] = (1.0 - s) * (1.0 - s)

    onehot = (jax.lax.broadcasted_iota(jnp.int32, (_B, _E), 1) == e)
    g_sel = jnp.sum(g * onehot.astype(jnp.float32), axis=1, keepdims=True)
    row_b = jax.lax.broadcasted_iota(jnp.int32, (_ROWS, _B), 0) // _L
    col_b = jax.lax.broadcasted_iota(jnp.int32, (_ROWS, _B), 1)
    P = (row_b == col_b).astype(jnp.float32)
    ge_col = jnp.dot(P, g_sel, preferred_element_type=jnp.float32)

    w = w_ref[0].astype(jnp.bfloat16)
    partial = jnp.dot(flat_ref[...], w, preferred_element_type=jnp.float32)
    contrib = ge_col * (partial + b_ref[0])

    @pl.when(e == 0)
    def _():
        acc_ref[...] = jnp.zeros_like(acc_ref)

    acc_ref[...] += contrib

    @pl.when(e == ne - 1)
    def _():
        out_ref[...] = acc_ref[...].reshape(_B, _L, _DM).astype(jnp.bfloat16)


def kernel(cycle_curve_data, logits, moe_masks, expert_w, expert_b):
    masks = moe_masks.astype(jnp.int32)

    out, gl = pl.pallas_call(
        _moe_kernel,
        grid=(_E,),
        in_specs=[
            pl.BlockSpec((_B, _E), lambda e: (0, 0)),
            pl.BlockSpec((_B, _E), lambda e: (0, 0)),
            pl.BlockSpec((_B, _L, 3, _CLEN), lambda e: (0, 0, 0, 0)),
            pl.BlockSpec((1, _FIN, _DM), lambda e: (e, 0, 0)),
            pl.BlockSpec((1, 1, _DM), lambda e: (e, 0, 0)),
        ],
        out_specs=[
            pl.BlockSpec((_B, _L, _DM), lambda e: (0, 0, 0)),
            pl.BlockSpec(memory_space=pltpu.SMEM),
        ],
        out_shape=[
            jax.ShapeDtypeStruct((_B, _L, _DM), jnp.bfloat16),
            jax.ShapeDtypeStruct((1, 1), jnp.float32),
        ],
        scratch_shapes=[
            pltpu.VMEM((_ROWS, _FIN), jnp.bfloat16),
            pltpu.VMEM((_ROWS, _DM), jnp.float32),
        ],
    )(logits, masks, cycle_curve_data, expert_w, expert_b.reshape(_E, 1, _DM))

    return out, gl.reshape(())


# speedup vs baseline: 1.3602x; 1.0416x over previous
import jax
import jax.numpy as jnp
from jax.experimental import pallas as pl
from jax.experimental.pallas import tpu as pltpu

_B, _L, _CLEN, _E, _DM = 16, 64, 512, 8, 1024
_FIN = 3 * _CLEN
_ROWS = _B * _L
_NJ = 2
_DT = _DM // _NJ


def _moe_kernel(logits_ref, masks_ref, cc_hbm, w_ref, b_ref, out_ref, gl_ref,
                flat32_ref, flatb_ref, acc_ref, sems):
    j = pl.program_id(0)
    e = pl.program_id(1)

    @pl.when((j == 0) & (e == 0))
    def _():
        for c in range(3):
            cp = pltpu.make_async_copy(
                cc_hbm.at[:, :, c, 0, :],
                flat32_ref.at[:, :, c * _CLEN:(c + 1) * _CLEN],
                sems.at[c],
            )
            cp.start()
        for c in range(3):
            pltpu.make_async_copy(
                cc_hbm.at[:, :, c, 0, :],
                flat32_ref.at[:, :, c * _CLEN:(c + 1) * _CLEN],
                sems.at[c],
            ).wait()
        flatb_ref[...] = flat32_ref[...].reshape(_ROWS, _FIN).astype(jnp.bfloat16)

    logits = logits_ref[...]
    mask = (masks_ref[...] == 1).astype(jnp.float32)
    m = jnp.max(logits, axis=1, keepdims=True)
    ex = jnp.exp(logits - m)
    sm = ex / jnp.sum(ex, axis=1, keepdims=True)
    g = sm * mask
    g = g / (jnp.sum(g, axis=1, keepdims=True) + 1e-9)

    @pl.when((j == 0) & (e == 0))
    def _():
        s = jnp.sum(sm * mask) / _B
        gl_ref[0, 0] = (1.0 - s) * (1.0 - s)

    onehot = (jax.lax.broadcasted_iota(jnp.int32, (_B, _E), 1) == e)
    g_sel = jnp.sum(g * onehot.astype(jnp.float32), axis=1, keepdims=True)
    row_b = jax.lax.broadcasted_iota(jnp.int32, (_ROWS, _B), 0) // _L
    col_b = jax.lax.broadcasted_iota(jnp.int32, (_ROWS, _B), 1)
    P = (row_b == col_b).astype(jnp.float32)
    ge_col = jnp.dot(P, g_sel, preferred_element_type=jnp.float32)

    w = w_ref[0].astype(jnp.bfloat16)
    partial = jnp.dot(flatb_ref[...], w, preferred_element_type=jnp.float32)
    contrib = ge_col * (partial + b_ref[0])

    @pl.when(e == 0)
    def _():
        acc_ref[...] = contrib

    @pl.when(e > 0)
    def _():
        acc_ref[...] += contrib

    @pl.when(e == _E - 1)
    def _():
        out_ref[...] = acc_ref[...].reshape(_B, _L, _DT).astype(jnp.bfloat16)


def kernel(cycle_curve_data, logits, moe_masks, expert_w, expert_b):
    masks = moe_masks.astype(jnp.int32)

    out, gl = pl.pallas_call(
        _moe_kernel,
        grid=(_NJ, _E),
        in_specs=[
            pl.BlockSpec((_B, _E), lambda j, e: (0, 0)),
            pl.BlockSpec((_B, _E), lambda j, e: (0, 0)),
            pl.BlockSpec(memory_space=pl.MemorySpace.ANY),
            pl.BlockSpec((1, _FIN, _DT), lambda j, e: (e, 0, j)),
            pl.BlockSpec((1, 1, _DT), lambda j, e: (e, 0, j)),
        ],
        out_specs=[
            pl.BlockSpec((_B, _L, _DT), lambda j, e: (0, 0, j)),
            pl.BlockSpec(memory_space=pltpu.SMEM),
        ],
        out_shape=[
            jax.ShapeDtypeStruct((_B, _L, _DM), jnp.bfloat16),
            jax.ShapeDtypeStruct((1, 1), jnp.float32),
        ],
        scratch_shapes=[
            pltpu.VMEM((_B, _L, _FIN), jnp.float32),
            pltpu.VMEM((_ROWS, _FIN), jnp.bfloat16),
            pltpu.VMEM((_ROWS, _DT), jnp.float32),
            pltpu.SemaphoreType.DMA((3,)),
        ],
    )(logits, masks, cycle_curve_data.reshape(_B, _L, 3, 1, _CLEN),
      expert_w, expert_b.reshape(_E, 1, _DM))

    return out, gl.reshape(())


# speedup vs baseline: 1.3682x; 1.0059x over previous
import jax
import jax.numpy as jnp
from jax.experimental import pallas as pl
from jax.experimental.pallas import tpu as pltpu

_B, _L, _CLEN, _E, _DM = 16, 64, 512, 8, 1024
_FIN = 3 * _CLEN
_ROWS = _B * _L
_NJ = 2
_DT = _DM // _NJ


def _moe_kernel(logits_ref, masks_ref, cc_hbm, w_ref, b_ref, out_ref, gl_ref,
                flat32_ref, acc_ref, sems):
    j = pl.program_id(0)
    e = pl.program_id(1)

    @pl.when((j == 0) & (e == 0))
    def _():
        for c in range(3):
            cp = pltpu.make_async_copy(
                cc_hbm.at[:, :, c, 0, :],
                flat32_ref.at[:, :, c * _CLEN:(c + 1) * _CLEN],
                sems.at[c],
            )
            cp.start()
        for c in range(3):
            pltpu.make_async_copy(
                cc_hbm.at[:, :, c, 0, :],
                flat32_ref.at[:, :, c * _CLEN:(c + 1) * _CLEN],
                sems.at[c],
            ).wait()

    logits = logits_ref[...]
    mask = (masks_ref[...] == 1).astype(jnp.float32)
    m = jnp.max(logits, axis=1, keepdims=True)
    ex = jnp.exp(logits - m)
    sm = ex / jnp.sum(ex, axis=1, keepdims=True)
    g = sm * mask
    g = g / (jnp.sum(g, axis=1, keepdims=True) + 1e-9)

    @pl.when((j == 0) & (e == 0))
    def _():
        s = jnp.sum(sm * mask) / _B
        gl_ref[0, 0] = (1.0 - s) * (1.0 - s)

    onehot = (jax.lax.broadcasted_iota(jnp.int32, (_B, _E), 1) == e)
    g_sel = jnp.sum(g * onehot.astype(jnp.float32), axis=1, keepdims=True)
    row_b = jax.lax.broadcasted_iota(jnp.int32, (_ROWS, _B), 0) // _L
    col_b = jax.lax.broadcasted_iota(jnp.int32, (_ROWS, _B), 1)
    P = (row_b == col_b).astype(jnp.float32)
    ge_col = jnp.dot(P, g_sel, preferred_element_type=jnp.float32)

    lhs = flat32_ref[...].reshape(_ROWS, _FIN)
    partial = jnp.dot(lhs, w_ref[0], preferred_element_type=jnp.float32)
    contrib = ge_col * (partial + b_ref[0])

    @pl.when(e == 0)
    def _():
        acc_ref[...] = contrib

    @pl.when(e > 0)
    def _():
        acc_ref[...] += contrib

    @pl.when(e == _E - 1)
    def _():
        out_ref[...] = acc_ref[...].reshape(_B, _L, _DT).astype(jnp.bfloat16)


def kernel(cycle_curve_data, logits, moe_masks, expert_w, expert_b):
    masks = moe_masks.astype(jnp.int32)

    out, gl = pl.pallas_call(
        _moe_kernel,
        grid=(_NJ, _E),
        in_specs=[
            pl.BlockSpec((_B, _E), lambda j, e: (0, 0)),
            pl.BlockSpec((_B, _E), lambda j, e: (0, 0)),
            pl.BlockSpec(memory_space=pl.MemorySpace.ANY),
            pl.BlockSpec((1, _FIN, _DT), lambda j, e: (e, 0, j)),
            pl.BlockSpec((1, 1, _DT), lambda j, e: (e, 0, j)),
        ],
        out_specs=[
            pl.BlockSpec((_B, _L, _DT), lambda j, e: (0, 0, j)),
            pl.BlockSpec(memory_space=pltpu.SMEM),
        ],
        out_shape=[
            jax.ShapeDtypeStruct((_B, _L, _DM), jnp.bfloat16),
            jax.ShapeDtypeStruct((1, 1), jnp.float32),
        ],
        scratch_shapes=[
            pltpu.VMEM((_B, _L, _FIN), jnp.float32),
            pltpu.VMEM((_ROWS, _DT), jnp.float32),
            pltpu.SemaphoreType.DMA((3,)),
        ],
    )(logits, masks, cycle_curve_data.reshape(_B, _L, 3, 1, _CLEN),
      expert_w, expert_b.reshape(_E, 1, _DM))

    return out, gl.reshape(())
